# Initial kernel scaffold; baseline (speedup 1.0000x reference)
import jax
import jax.numpy as jnp
from jax.experimental import pallas as pl


def kernel(x, edge_index, weight, W1, b1, W2, b2, W3, b3, A2w, A2b):
    raise NotImplementedError("write your pallas kernel here")


# speedup vs baseline: 6.5561x; 6.5561x over previous
import functools

import jax
import jax.numpy as jnp
from jax import lax
from jax.experimental import pallas as pl
from jax.experimental.pallas import tpu as pltpu
from jax.experimental.pallas import tpu_sc as plsc

N = 10000
E = 320000
D = 128
H = 16

NC = 2
NS = 16
NW = NC * NS
BLK = 128
EP = ((E + NW * BLK - 1) // (NW * BLK)) * (NW * BLK)
R = EP // BLK
RPT = R // NW
NPS = N // NS
N2 = 10240
NPS2 = N2 // NS

_mesh = functools.partial(
    plsc.VectorSubcoreMesh, core_axis_name="c", subcore_axis_name="s")


def _wid():
    return lax.axis_index("c") * NS + lax.axis_index("s")


def _deg_call(src2d, w2d, z1):
    @functools.partial(
        pl.kernel,
        out_type=jax.ShapeDtypeStruct((NC, N2), jnp.float32),
        mesh=_mesh(),
        compiler_params=pltpu.CompilerParams(needs_layout_passes=False, use_tc_tiling_on_sc=False),
        scratch_types=[
            pltpu.VMEM_SHARED((N2,), jnp.float32),
            pltpu.VMEM((BLK,), jnp.int32),
            pltpu.VMEM((BLK,), jnp.float32),
        ],
    )
    def k(src_h, w_h, z_h, out_h, shared, idx_v, val_v):
        c = lax.axis_index("c")
        s = lax.axis_index("s")
        w = _wid()
        pltpu.sync_copy(z_h.at[pl.ds(s * NPS2, NPS2)],
                        shared.at[pl.ds(s * NPS2, NPS2)])
        plsc.subcore_barrier()

        def body(r, _):
            row = w * RPT + r
            pltpu.sync_copy(src_h.at[row], idx_v)
            pltpu.sync_copy(w_h.at[row], val_v)
            pltpu.sync_copy(val_v, shared.at[idx_v], add=True)
            return _

        lax.fori_loop(0, RPT, body, None)
        plsc.subcore_barrier()
        pltpu.sync_copy(shared.at[pl.ds(s * NPS2, NPS2)],
                        out_h.at[c, pl.ds(s * NPS2, NPS2)])

    return k(src2d, w2d, z1)


def _tc_dis(deg_p):
    def body(deg_ref, dis_ref):
        d = deg_ref[0, :] + deg_ref[1, :]
        dis_ref[...] = jnp.where(d > 0.0, lax.rsqrt(d), 0.0)

    return pl.pallas_call(
        body,
        out_shape=jax.ShapeDtypeStruct((N2,), jnp.float32),
    )(deg_p)


def _norm_call(dis1d, src2d, dst2d, w2d):
    @functools.partial(
        pl.kernel,
        out_type=jax.ShapeDtypeStruct((R, BLK), jnp.float32),
        mesh=_mesh(),
        compiler_params=pltpu.CompilerParams(needs_layout_passes=False, use_tc_tiling_on_sc=False),
        scratch_types=[
            pltpu.VMEM((N2,), jnp.float32),
            pltpu.VMEM((BLK,), jnp.int32),
            pltpu.VMEM((BLK,), jnp.int32),
            pltpu.VMEM((BLK,), jnp.float32),
            pltpu.VMEM((BLK,), jnp.float32),
        ],
    )
    def k(dis_h, src_h, dst_h, w_h, out_h, dis_v, si_v, di_v, wv, ov):
        w = _wid()
        pltpu.sync_copy(dis_h, dis_v)

        def body(r, _):
            row = w * RPT + r
            pltpu.sync_copy(src_h.at[row], si_v)
            pltpu.sync_copy(dst_h.at[row], di_v)
            pltpu.sync_copy(w_h.at[row], wv)
            for j in range(BLK // 16):
                sl = pl.ds(j * 16, 16)
                ds_ = plsc.load_gather(dis_v, [si_v[sl]])
                dd_ = plsc.load_gather(dis_v, [di_v[sl]])
                ov[sl] = -(ds_ * wv[sl] * dd_)
            pltpu.sync_copy(ov, out_h.at[row])
            return _

        lax.fori_loop(0, RPT, body, None)

    return k(dis1d, src2d, dst2d, w2d)


def _prop_call(table, src2d, dst2d, nn2d, zF, F):
    @functools.partial(
        pl.kernel,
        out_type=jax.ShapeDtypeStruct((NC, N2, F), jnp.float32),
        mesh=_mesh(),
        compiler_params=pltpu.CompilerParams(needs_layout_passes=False, use_tc_tiling_on_sc=False),
        scratch_types=[
            pltpu.VMEM_SHARED((N2, F), jnp.float32),
            pltpu.VMEM((BLK,), jnp.int32),
            pltpu.VMEM((BLK,), jnp.int32),
            pltpu.VMEM((BLK,), jnp.float32),
            pltpu.VMEM((BLK, F), jnp.float32),
            pltpu.SemaphoreType.DMA,
        ],
    )
    def k(tab_h, src_h, dst_h, nn_h, z_h, out_h,
          shared, si_v, di_v, nn_v, rows_v, sem):
        c = lax.axis_index("c")
        s = lax.axis_index("s")
        w = _wid()
        pltpu.sync_copy(z_h.at[pl.ds(s * NPS2, NPS2)],
                        shared.at[pl.ds(s * NPS2, NPS2)])
        plsc.subcore_barrier()

        def body(r, _):
            row = w * RPT + r
            pltpu.sync_copy(src_h.at[row], si_v)
            pltpu.sync_copy(dst_h.at[row], di_v)
            pltpu.sync_copy(nn_h.at[row], nn_v)
            pltpu.async_copy(tab_h.at[si_v], rows_v, sem).wait()

            def scale(g, _):
                base = g * 16
                nnvec = nn_v[pl.ds(base, 16)]
                for i in range(16):
                    sc = nnvec[i]
                    for j in range(F // 16):
                        sl = pl.ds(j * 16, 16)
                        rows_v[base + i, sl] = rows_v[base + i, sl] * sc
                return _

            lax.fori_loop(0, BLK // 16, scale, None)
            pltpu.sync_copy(rows_v, shared.at[di_v], add=True)
            return _

        lax.fori_loop(0, RPT, body, None)
        plsc.subcore_barrier()
        pltpu.sync_copy(shared.at[pl.ds(s * NPS2, NPS2)],
                        out_h.at[c, pl.ds(s * NPS2, NPS2)])

    return k(table, src2d, dst2d, nn2d, zF)


def _prop2_call(p1a, p1b, src2d, dst2d, nn2d, z16):
    @functools.partial(
        pl.kernel,
        out_type=jax.ShapeDtypeStruct((NC, N2, 16), jnp.float32),
        mesh=_mesh(),
        compiler_params=pltpu.CompilerParams(needs_layout_passes=False, use_tc_tiling_on_sc=False),
        scratch_types=[
            pltpu.VMEM_SHARED((N2, 16), jnp.float32),
            pltpu.VMEM((BLK,), jnp.int32),
            pltpu.VMEM((BLK,), jnp.int32),
            pltpu.VMEM((BLK,), jnp.float32),
            pltpu.VMEM((BLK, 32), jnp.float32),
            pltpu.VMEM((BLK, 32), jnp.float32),
            pltpu.VMEM((BLK, 16), jnp.float32),
            pltpu.SemaphoreType.DMA,
        ],
    )
    def k(pa_h, pb_h, src_h, dst_h, nn_h, z_h, out_h,
          shared, si_v, di_v, nn_v, ra_v, rb_v, msg_v, sem):
        c = lax.axis_index("c")
        s = lax.axis_index("s")
        w = _wid()
        pltpu.sync_copy(z_h.at[pl.ds(s * NPS2, NPS2)],
                        shared.at[pl.ds(s * NPS2, NPS2)])
        plsc.subcore_barrier()

        def body(r, _):
            row = w * RPT + r
            pltpu.sync_copy(src_h.at[row], si_v)
            pltpu.sync_copy(dst_h.at[row], di_v)
            pltpu.sync_copy(nn_h.at[row], nn_v)
            cp_a = pltpu.async_copy(pa_h.at[si_v], ra_v, sem)
            cp_b = pltpu.async_copy(pb_h.at[si_v], rb_v, sem)
            cp_a.wait()
            cp_b.wait()

            def scale(g, _):
                base = g * 16
                nnvec = nn_v[pl.ds(base, 16)]
                sl = pl.ds(16, 16)
                for i in range(16):
                    sc = nnvec[i]
                    msg_v[base + i, :] = (ra_v[base + i, sl]
                                          + rb_v[base + i, sl]) * sc
                return _

            lax.fori_loop(0, BLK // 16, scale, None)
            pltpu.sync_copy(msg_v, shared.at[di_v], add=True)
            return _

        lax.fori_loop(0, RPT, body, None)
        plsc.subcore_barrier()
        pltpu.sync_copy(shared.at[pl.ds(s * NPS2, NPS2)],
                        out_h.at[c, pl.ds(s * NPS2, NPS2)])

    return k(p1a, p1b, src2d, dst2d, nn2d, z16)


def _tc_in(x, Wcat):
    def body(x_ref, w_ref, zA_ref, zBC_ref):
        h = lax.dot_general(x_ref[...], w_ref[...], (((1,), (0,)), ((), ())),
                            precision=lax.Precision.HIGHEST,
                            preferred_element_type=jnp.float32)
        zA_ref[...] = h[:, :16]
        zBC_ref[...] = h[:, 16:48]

    BN = N2 // 8
    return pl.pallas_call(
        body,
        grid=(8,),
        in_specs=[pl.BlockSpec((BN, D), lambda i: (i, 0)),
                  pl.BlockSpec((D, 48), lambda i: (0, 0))],
        out_specs=[pl.BlockSpec((BN, 16), lambda i: (i, 0)),
                   pl.BlockSpec((BN, 32), lambda i: (i, 0))],
        out_shape=(jax.ShapeDtypeStruct((N2, 16), jnp.float32),
                   jax.ShapeDtypeStruct((N2, 32), jnp.float32)),
    )(x, Wcat)


def _tc_combine(zA, p1a, p1b, p2a, p2b, brow, Wcat):
    def body(zA_ref, p1a_ref, p1b_ref, p2a_ref, p2b_ref, b_ref, w_ref,
             zA2_ref, zBC2_ref, h_ref):
        act = (zA_ref[...] + p1a_ref[:, :16] + p1b_ref[:, :16]
               + 2.0 * (p2a_ref[...] + p2b_ref[...]) + b_ref[...])
        act = jnp.maximum(act, 0.0)
        h_ref[...] = act
        z = lax.dot_general(act, w_ref[...], (((1,), (0,)), ((), ())),
                            precision=lax.Precision.HIGHEST,
                            preferred_element_type=jnp.float32)
        zA2_ref[...] = z[:, :16]
        zBC2_ref[...] = z[:, 16:48]

    BN = N2 // 8
    return pl.pallas_call(
        body,
        grid=(8,),
        in_specs=[pl.BlockSpec((BN, 16), lambda i: (i, 0)),
                  pl.BlockSpec((BN, 32), lambda i: (i, 0)),
                  pl.BlockSpec((BN, 32), lambda i: (i, 0)),
                  pl.BlockSpec((BN, 16), lambda i: (i, 0)),
                  pl.BlockSpec((BN, 16), lambda i: (i, 0)),
                  pl.BlockSpec((1, 16), lambda i: (0, 0)),
                  pl.BlockSpec((16, 48), lambda i: (0, 0))],
        out_specs=[pl.BlockSpec((BN, 16), lambda i: (i, 0)),
                   pl.BlockSpec((BN, 32), lambda i: (i, 0)),
                   pl.BlockSpec((BN, 16), lambda i: (i, 0))],
        out_shape=(jax.ShapeDtypeStruct((N2, 16), jnp.float32),
                   jax.ShapeDtypeStruct((N2, 32), jnp.float32),
                   jax.ShapeDtypeStruct((N2, 16), jnp.float32)),
    )(zA, p1a, p1b, p2a, p2b, brow, Wcat)


def _tc_final(zA3, p1a, p1b, p2a, p2b, brow, h2, A2w, A2b):
    def body(zA_ref, p1a_ref, p1b_ref, p2a_ref, p2b_ref, b_ref, h2_ref,
             aw_ref, ab_ref, choice_ref, value_ref):
        cfull = (zA_ref[...] + p1a_ref[:, :16] + p1b_ref[:, :16]
                 + 2.0 * (p2a_ref[...] + p2b_ref[...]) + b_ref[...])
        valid = lax.broadcasted_iota(jnp.int32, (N2, 1), 0) < N
        c = jnp.where(valid, cfull[:, 0:1], -jnp.inf)
        m = jnp.max(c)
        ex = jnp.exp(c - m)
        choice_ref[...] = ex / jnp.sum(ex)
        v = jnp.sum(jnp.where(valid, h2_ref[...], 0.0), axis=0,
                    keepdims=True) * (1.0 / N)
        value_ref[...] = (
            jnp.sum(v * aw_ref[...], axis=1, keepdims=True) + ab_ref[...])

    return pl.pallas_call(
        body,
        out_shape=(jax.ShapeDtypeStruct((N2, 1), jnp.float32),
                   jax.ShapeDtypeStruct((1, 1), jnp.float32)),
    )(zA3, p1a, p1b, p2a, p2b, brow, h2, A2w, A2b)


def kernel(x, edge_index, weight, W1, b1, W2, b2, W3, b3, A2w, A2b):
    pad = EP - E
    src = jnp.pad(edge_index[0], (0, pad)).reshape(R, BLK)
    dst = jnp.pad(edge_index[1], (0, pad)).reshape(R, BLK)
    w2d = jnp.pad(weight, (0, pad)).reshape(R, BLK)

    W1cat = jnp.concatenate([W1[0] - W1[2], W1[1], W1[2]], axis=1)
    W2cat = jnp.concatenate([W2[0] - W2[2], W2[1], W2[2]], axis=1)
    W3p = jnp.pad(W3, ((0, 0), (0, 0), (0, 15)))
    W3cat = jnp.concatenate([W3p[0] - W3p[2], W3p[1], W3p[2]], axis=1)
    b1r = b1.reshape(1, 16)
    b2r = b2.reshape(1, 16)
    b3r = jnp.pad(b3, (0, 15)).reshape(1, 16)

    z1 = jnp.zeros((N2,), jnp.float32)
    z16 = jnp.zeros((N2, 16), jnp.float32)
    z32 = jnp.zeros((N2, 32), jnp.float32)

    deg_p = _deg_call(src, w2d, z1)
    dis1d = _tc_dis(deg_p)
    nn2d = _norm_call(dis1d, src, dst, w2d)

    xp = jnp.pad(x, ((0, N2 - N), (0, 0)))
    zA, zBC = _tc_in(xp, W1cat)
    for layer in range(3):
        p1 = _prop_call(zBC, src, dst, nn2d, z32, 32)
        p1a, p1b = p1[0], p1[1]
        p2 = _prop2_call(p1a, p1b, src, dst, nn2d, z16)
        if layer == 0:
            zA, zBC, _ = _tc_combine(zA, p1a, p1b, p2[0], p2[1], b1r, W2cat)
        elif layer == 1:
            zA, zBC, h2 = _tc_combine(zA, p1a, p1b, p2[0], p2[1], b2r, W3cat)
        else:
            choice, value = _tc_final(zA, p1a, p1b, p2[0], p2[1], b3r, h2,
                                      A2w, A2b.reshape(1, 1))
    return choice[:N, 0], value.reshape(())


# speedup vs baseline: 16.1716x; 2.4667x over previous
import functools

import jax
import jax.numpy as jnp
from jax import lax
from jax.experimental import pallas as pl
from jax.experimental.pallas import tpu as pltpu
from jax.experimental.pallas import tpu_sc as plsc

N = 10000
E = 320000
D = 128
H = 16

NC = 2
NS = 16
NW = NC * NS
BLK = 128
EP = ((E + NW * BLK - 1) // (NW * BLK)) * (NW * BLK)
R = EP // BLK
RPT = R // NW
N2 = 10240
NPS2 = N2 // NS

_mesh = functools.partial(
    plsc.VectorSubcoreMesh, core_axis_name="c", subcore_axis_name="s")

_sc_params = functools.partial(
    pltpu.CompilerParams, needs_layout_passes=False, use_tc_tiling_on_sc=False)


def _wid():
    return lax.axis_index("c") * NS + lax.axis_index("s")


def _deg_call(src2d, w2d, z1):
    @functools.partial(
        pl.kernel,
        out_type=jax.ShapeDtypeStruct((NC, N2), jnp.float32),
        mesh=_mesh(),
        compiler_params=_sc_params(),
        scratch_types=[
            pltpu.VMEM_SHARED((N2,), jnp.float32),
            pltpu.VMEM((RPT, BLK), jnp.int32),
            pltpu.VMEM((RPT, BLK), jnp.float32),
            pltpu.VMEM((BLK,), jnp.float32),
            pltpu.SemaphoreType.DMA,
        ],
    )
    def k(src_h, w_h, z_h, out_h, shared, si_a, w_a, drow, sem):
        c = lax.axis_index("c")
        s = lax.axis_index("s")
        base = _wid() * RPT
        pltpu.sync_copy(z_h.at[pl.ds(s * NPS2, NPS2)],
                        shared.at[pl.ds(s * NPS2, NPS2)])
        pltpu.sync_copy(src_h.at[pl.ds(base, RPT)], si_a)
        pltpu.sync_copy(w_h.at[pl.ds(base, RPT)], w_a)
        plsc.subcore_barrier()

        def issue(r, _):
            pltpu.async_copy(w_a.at[r], shared.at[si_a.at[r]], sem, add=True)
            return _

        lax.fori_loop(0, RPT, issue, None)

        def drain(r, _):
            pltpu.make_async_copy(z_h.at[pl.ds(0, BLK)], drow, sem).wait()
            return _

        lax.fori_loop(0, RPT, drain, None)
        plsc.subcore_barrier()
        pltpu.sync_copy(shared.at[pl.ds(s * NPS2, NPS2)],
                        out_h.at[c, pl.ds(s * NPS2, NPS2)])

    return k(src2d, w2d, z1)


def _tc_dis(deg_p):
    def body(deg_ref, dis_ref):
        d = deg_ref[0, :] + deg_ref[1, :]
        dis_ref[...] = jnp.where(d > 0.0, lax.rsqrt(d), 0.0)

    return pl.pallas_call(
        body,
        out_shape=jax.ShapeDtypeStruct((N2,), jnp.float32),
    )(deg_p)


def _norm_call(dis1d, src2d, dst2d, w2d):
    @functools.partial(
        pl.kernel,
        out_type=jax.ShapeDtypeStruct((R, BLK), jnp.float32),
        mesh=_mesh(),
        compiler_params=_sc_params(),
        scratch_types=[
            pltpu.VMEM((N2,), jnp.float32),
            pltpu.VMEM((RPT, BLK), jnp.int32),
            pltpu.VMEM((RPT, BLK), jnp.int32),
            pltpu.VMEM((RPT, BLK), jnp.float32),
            pltpu.VMEM((RPT, BLK), jnp.float32),
        ],
    )
    def k(dis_h, src_h, dst_h, w_h, out_h, dis_v, si_a, di_a, w_a, o_a):
        base = _wid() * RPT
        pltpu.sync_copy(dis_h, dis_v)
        pltpu.sync_copy(src_h.at[pl.ds(base, RPT)], si_a)
        pltpu.sync_copy(dst_h.at[pl.ds(base, RPT)], di_a)
        pltpu.sync_copy(w_h.at[pl.ds(base, RPT)], w_a)

        def body(r, _):
            for j in range(BLK // 16):
                sl = pl.ds(j * 16, 16)
                ds_ = plsc.load_gather(dis_v, [si_a[r, sl]])
                dd_ = plsc.load_gather(dis_v, [di_a[r, sl]])
                o_a[r, sl] = -(ds_ * w_a[r, sl] * dd_)
            return _

        lax.fori_loop(0, RPT, body, None)
        pltpu.sync_copy(o_a, out_h.at[pl.ds(base, RPT)])

    return k(dis1d, src2d, dst2d, w2d)


def _prop_call(table, src2d, dst2d, nn2d, zF, F):
    @functools.partial(
        pl.kernel,
        out_type=jax.ShapeDtypeStruct((NC, N2, F), jnp.float32),
        mesh=_mesh(),
        compiler_params=_sc_params(),
        scratch_types=[
            pltpu.VMEM_SHARED((N2, F), jnp.float32),
            pltpu.VMEM((RPT, BLK), jnp.int32),
            pltpu.VMEM((RPT, BLK), jnp.int32),
            pltpu.VMEM((RPT, BLK), jnp.float32),
            pltpu.VMEM((2, BLK, F), jnp.float32),
            pltpu.VMEM((2, BLK, F), jnp.float32),
            pltpu.SemaphoreType.DMA,
            pltpu.SemaphoreType.DMA,
            pltpu.SemaphoreType.DMA,
            pltpu.SemaphoreType.DMA,
        ],
    )
    def k(tab_h, src_h, dst_h, nn_h, z_h, out_h,
          shared, si_a, di_a, nn_a, gbuf, sbuf, sg0, sg1, ss0, ss1):
        c = lax.axis_index("c")
        s = lax.axis_index("s")
        base = _wid() * RPT
        pltpu.sync_copy(z_h.at[pl.ds(s * NPS2, NPS2)],
                        shared.at[pl.ds(s * NPS2, NPS2)])
        pltpu.sync_copy(src_h.at[pl.ds(base, RPT)], si_a)
        pltpu.sync_copy(dst_h.at[pl.ds(base, RPT)], di_a)
        pltpu.sync_copy(nn_h.at[pl.ds(base, RPT)], nn_a)
        plsc.subcore_barrier()

        sems_g = (sg0, sg1)
        sems_s = (ss0, ss1)

        def stage(r, cur, nxt):
            @pl.when(r + 1 < RPT)
            def _():
                pltpu.async_copy(tab_h.at[si_a.at[r + 1]], gbuf.at[nxt],
                                 sems_g[nxt])

            pltpu.make_async_copy(tab_h.at[pl.ds(0, BLK)], gbuf.at[cur],
                                  sems_g[cur]).wait()

            @pl.when(r >= 2)
            def _():
                pltpu.make_async_copy(z_h.at[pl.ds(0, BLK)], sbuf.at[cur],
                                      sems_s[cur]).wait()

            def scale(g, _):
                b16 = g * 16
                nnvec = nn_a[r, pl.ds(b16, 16)]
                for i in range(16):
                    sc = nnvec[i]
                    for j in range(F // 16):
                        sl = pl.ds(j * 16, 16)
                        sbuf[cur, b16 + i, sl] = gbuf[cur, b16 + i, sl] * sc
                return _

            lax.fori_loop(0, BLK // 16, scale, None)
            pltpu.async_copy(sbuf.at[cur], shared.at[di_a.at[r]],
                             sems_s[cur], add=True)

        pltpu.async_copy(tab_h.at[si_a.at[0]], gbuf.at[0], sg0)

        def body(r, _):
            @pl.when(lax.rem(r, 2) == 0)
            def _():
                stage(r, 0, 1)

            @pl.when(lax.rem(r, 2) == 1)
            def _():
                stage(r, 1, 0)

            return _

        lax.fori_loop(0, RPT, body, None)
        pltpu.make_async_copy(z_h.at[pl.ds(0, BLK)], sbuf.at[(RPT - 2) % 2],
                              sems_s[(RPT - 2) % 2]).wait()
        pltpu.make_async_copy(z_h.at[pl.ds(0, BLK)], sbuf.at[(RPT - 1) % 2],
                              sems_s[(RPT - 1) % 2]).wait()
        plsc.subcore_barrier()
        pltpu.sync_copy(shared.at[pl.ds(s * NPS2, NPS2)],
                        out_h.at[c, pl.ds(s * NPS2, NPS2)])

    return k(table, src2d, dst2d, nn2d, zF)


def _prop2_call(p1a, p1b, src2d, dst2d, nn2d, z16):
    @functools.partial(
        pl.kernel,
        out_type=jax.ShapeDtypeStruct((NC, N2, 16), jnp.float32),
        mesh=_mesh(),
        compiler_params=_sc_params(),
        scratch_types=[
            pltpu.VMEM_SHARED((N2, 16), jnp.float32),
            pltpu.VMEM((RPT, BLK), jnp.int32),
            pltpu.VMEM((RPT, BLK), jnp.int32),
            pltpu.VMEM((RPT, BLK), jnp.float32),
            pltpu.VMEM((2, BLK, 32), jnp.float32),
            pltpu.VMEM((2, BLK, 32), jnp.float32),
            pltpu.VMEM((2, BLK, 16), jnp.float32),
            pltpu.SemaphoreType.DMA,
            pltpu.SemaphoreType.DMA,
            pltpu.SemaphoreType.DMA,
            pltpu.SemaphoreType.DMA,
        ],
    )
    def k(pa_h, pb_h, src_h, dst_h, nn_h, z_h, out_h,
          shared, si_a, di_a, nn_a, ga, gb, sbuf, sg0, sg1, ss0, ss1):
        c = lax.axis_index("c")
        s = lax.axis_index("s")
        base = _wid() * RPT
        pltpu.sync_copy(z_h.at[pl.ds(s * NPS2, NPS2)],
                        shared.at[pl.ds(s * NPS2, NPS2)])
        pltpu.sync_copy(src_h.at[pl.ds(base, RPT)], si_a)
        pltpu.sync_copy(dst_h.at[pl.ds(base, RPT)], di_a)
        pltpu.sync_copy(nn_h.at[pl.ds(base, RPT)], nn_a)
        plsc.subcore_barrier()

        sems_g = (sg0, sg1)
        sems_s = (ss0, ss1)

        def issue_gathers(r, buf):
            pltpu.async_copy(pa_h.at[si_a.at[r]], ga.at[buf], sems_g[buf])
            pltpu.async_copy(pb_h.at[si_a.at[r]], gb.at[buf], sems_g[buf])

        def stage(r, cur, nxt):
            @pl.when(r + 1 < RPT)
            def _():
                issue_gathers(r + 1, nxt)

            pltpu.make_async_copy(pa_h.at[pl.ds(0, BLK)], ga.at[cur],
                                  sems_g[cur]).wait()
            pltpu.make_async_copy(pa_h.at[pl.ds(0, BLK)], gb.at[cur],
                                  sems_g[cur]).wait()

            @pl.when(r >= 2)
            def _():
                pltpu.make_async_copy(z_h.at[pl.ds(0, BLK)], sbuf.at[cur],
                                      sems_s[cur]).wait()

            def scale(g, _):
                b16 = g * 16
                nnvec = nn_a[r, pl.ds(b16, 16)]
                shi = pl.ds(16, 16)
                for i in range(16):
                    sc = nnvec[i]
                    sbuf[cur, b16 + i, :] = (ga[cur, b16 + i, shi]
                                             + gb[cur, b16 + i, shi]) * sc
                return _

            lax.fori_loop(0, BLK // 16, scale, None)
            pltpu.async_copy(sbuf.at[cur], shared.at[di_a.at[r]],
                             sems_s[cur], add=True)

        issue_gathers(0, 0)

        def body(r, _):
            @pl.when(lax.rem(r, 2) == 0)
            def _():
                stage(r, 0, 1)

            @pl.when(lax.rem(r, 2) == 1)
            def _():
                stage(r, 1, 0)

            return _

        lax.fori_loop(0, RPT, body, None)
        pltpu.make_async_copy(z_h.at[pl.ds(0, BLK)], sbuf.at[(RPT - 2) % 2],
                              sems_s[(RPT - 2) % 2]).wait()
        pltpu.make_async_copy(z_h.at[pl.ds(0, BLK)], sbuf.at[(RPT - 1) % 2],
                              sems_s[(RPT - 1) % 2]).wait()
        plsc.subcore_barrier()
        pltpu.sync_copy(shared.at[pl.ds(s * NPS2, NPS2)],
                        out_h.at[c, pl.ds(s * NPS2, NPS2)])

    return k(p1a, p1b, src2d, dst2d, nn2d, z16)


def _tc_in(x, Wcat):
    def body(x_ref, w_ref, zA_ref, zBC_ref):
        h = lax.dot_general(x_ref[...], w_ref[...], (((1,), (0,)), ((), ())),
                            precision=lax.Precision.HIGHEST,
                            preferred_element_type=jnp.float32)
        zA_ref[...] = h[:, :16]
        zBC_ref[...] = h[:, 16:48]

    BN = N2 // 8
    return pl.pallas_call(
        body,
        grid=(8,),
        in_specs=[pl.BlockSpec((BN, D), lambda i: (i, 0)),
                  pl.BlockSpec((D, 48), lambda i: (0, 0))],
        out_specs=[pl.BlockSpec((BN, 16), lambda i: (i, 0)),
                   pl.BlockSpec((BN, 32), lambda i: (i, 0))],
        out_shape=(jax.ShapeDtypeStruct((N2, 16), jnp.float32),
                   jax.ShapeDtypeStruct((N2, 32), jnp.float32)),
    )(x, Wcat)


def _tc_combine(zA, p1a, p1b, p2a, p2b, brow, Wcat):
    def body(zA_ref, p1a_ref, p1b_ref, p2a_ref, p2b_ref, b_ref, w_ref,
             zA2_ref, zBC2_ref, h_ref):
        act = (zA_ref[...] + p1a_ref[:, :16] + p1b_ref[:, :16]
               + 2.0 * (p2a_ref[...] + p2b_ref[...]) + b_ref[...])
        act = jnp.maximum(act, 0.0)
        h_ref[...] = act
        z = lax.dot_general(act, w_ref[...], (((1,), (0,)), ((), ())),
                            precision=lax.Precision.HIGHEST,
                            preferred_element_type=jnp.float32)
        zA2_ref[...] = z[:, :16]
        zBC2_ref[...] = z[:, 16:48]

    BN = N2 // 8
    return pl.pallas_call(
        body,
        grid=(8,),
        in_specs=[pl.BlockSpec((BN, 16), lambda i: (i, 0)),
                  pl.BlockSpec((BN, 32), lambda i: (i, 0)),
                  pl.BlockSpec((BN, 32), lambda i: (i, 0)),
                  pl.BlockSpec((BN, 16), lambda i: (i, 0)),
                  pl.BlockSpec((BN, 16), lambda i: (i, 0)),
                  pl.BlockSpec((1, 16), lambda i: (0, 0)),
                  pl.BlockSpec((16, 48), lambda i: (0, 0))],
        out_specs=[pl.BlockSpec((BN, 16), lambda i: (i, 0)),
                   pl.BlockSpec((BN, 32), lambda i: (i, 0)),
                   pl.BlockSpec((BN, 16), lambda i: (i, 0))],
        out_shape=(jax.ShapeDtypeStruct((N2, 16), jnp.float32),
                   jax.ShapeDtypeStruct((N2, 32), jnp.float32),
                   jax.ShapeDtypeStruct((N2, 16), jnp.float32)),
    )(zA, p1a, p1b, p2a, p2b, brow, Wcat)


def _tc_final(zA3, p1a, p1b, p2a, p2b, brow, h2, A2w, A2b):
    def body(zA_ref, p1a_ref, p1b_ref, p2a_ref, p2b_ref, b_ref, h2_ref,
             aw_ref, ab_ref, choice_ref, value_ref):
        cfull = (zA_ref[...] + p1a_ref[:, :16] + p1b_ref[:, :16]
                 + 2.0 * (p2a_ref[...] + p2b_ref[...]) + b_ref[...])
        valid = lax.broadcasted_iota(jnp.int32, (N2, 1), 0) < N
        c = jnp.where(valid, cfull[:, 0:1], -jnp.inf)
        m = jnp.max(c)
        ex = jnp.exp(c - m)
        choice_ref[...] = ex / jnp.sum(ex)
        v = jnp.sum(jnp.where(valid, h2_ref[...], 0.0), axis=0,
                    keepdims=True) * (1.0 / N)
        value_ref[...] = (
            jnp.sum(v * aw_ref[...], axis=1, keepdims=True) + ab_ref[...])

    return pl.pallas_call(
        body,
        out_shape=(jax.ShapeDtypeStruct((N2, 1), jnp.float32),
                   jax.ShapeDtypeStruct((1, 1), jnp.float32)),
    )(zA3, p1a, p1b, p2a, p2b, brow, h2, A2w, A2b)


def kernel(x, edge_index, weight, W1, b1, W2, b2, W3, b3, A2w, A2b):
    pad = EP - E
    src = jnp.pad(edge_index[0], (0, pad)).reshape(R, BLK)
    dst = jnp.pad(edge_index[1], (0, pad)).reshape(R, BLK)
    w2d = jnp.pad(weight, (0, pad)).reshape(R, BLK)

    W1cat = jnp.concatenate([W1[0] - W1[2], W1[1], W1[2]], axis=1)
    W2cat = jnp.concatenate([W2[0] - W2[2], W2[1], W2[2]], axis=1)
    W3p = jnp.pad(W3, ((0, 0), (0, 0), (0, 15)))
    W3cat = jnp.concatenate([W3p[0] - W3p[2], W3p[1], W3p[2]], axis=1)
    b1r = b1.reshape(1, 16)
    b2r = b2.reshape(1, 16)
    b3r = jnp.pad(b3, (0, 15)).reshape(1, 16)

    z1 = jnp.zeros((N2,), jnp.float32)
    z16 = jnp.zeros((N2, 16), jnp.float32)
    z32 = jnp.zeros((N2, 32), jnp.float32)

    deg_p = _deg_call(src, w2d, z1)
    dis1d = _tc_dis(deg_p)
    nn2d = _norm_call(dis1d, src, dst, w2d)

    xp = jnp.pad(x, ((0, N2 - N), (0, 0)))
    zA, zBC = _tc_in(xp, W1cat)
    for layer in range(3):
        p1 = _prop_call(zBC, src, dst, nn2d, z32, 32)
        p1a, p1b = p1[0], p1[1]
        p2 = _prop2_call(p1a, p1b, src, dst, nn2d, z16)
        if layer == 0:
            zA, zBC, _ = _tc_combine(zA, p1a, p1b, p2[0], p2[1], b1r, W2cat)
        elif layer == 1:
            zA, zBC, h2 = _tc_combine(zA, p1a, p1b, p2[0], p2[1], b2r, W3cat)
        else:
            choice, value = _tc_final(zA, p1a, p1b, p2[0], p2[1], b3r, h2,
                                      A2w, A2b.reshape(1, 1))
    return choice[:N, 0], value.reshape(())


# speedup vs baseline: 16.1939x; 1.0014x over previous
import functools

import jax
import jax.numpy as jnp
from jax import lax
from jax.experimental import pallas as pl
from jax.experimental.pallas import tpu as pltpu
from jax.experimental.pallas import tpu_sc as plsc

N = 10000
E = 320000
D = 128
H = 16

NC = 2
NS = 16
NW = NC * NS
BLK = 128
EP = ((E + NW * BLK - 1) // (NW * BLK)) * (NW * BLK)
R = EP // BLK
RPT = R // NW
N2 = 10240
NPS2 = N2 // NS

_mesh = functools.partial(
    plsc.VectorSubcoreMesh, core_axis_name="c", subcore_axis_name="s")

_sc_params = functools.partial(
    pltpu.CompilerParams, needs_layout_passes=False, use_tc_tiling_on_sc=False)


def _wid():
    return lax.axis_index("c") * NS + lax.axis_index("s")


def _deg_call(src2d, w2d, z1):
    @functools.partial(
        pl.kernel,
        out_type=jax.ShapeDtypeStruct((NC, N2), jnp.float32),
        mesh=_mesh(),
        compiler_params=_sc_params(),
        scratch_types=[
            pltpu.VMEM_SHARED((N2,), jnp.float32),
            pltpu.VMEM((RPT, BLK), jnp.int32),
            pltpu.VMEM((RPT, BLK), jnp.float32),
            pltpu.VMEM((BLK,), jnp.float32),
            pltpu.SemaphoreType.DMA,
        ],
    )
    def k(src_h, w_h, z_h, out_h, shared, si_a, w_a, drow, sem):
        c = lax.axis_index("c")
        s = lax.axis_index("s")
        base = _wid() * RPT
        pltpu.sync_copy(z_h.at[pl.ds(s * NPS2, NPS2)],
                        shared.at[pl.ds(s * NPS2, NPS2)])
        pltpu.sync_copy(src_h.at[pl.ds(base, RPT)], si_a)
        pltpu.sync_copy(w_h.at[pl.ds(base, RPT)], w_a)
        plsc.subcore_barrier()

        def issue(r, _):
            pltpu.async_copy(w_a.at[r], shared.at[si_a.at[r]], sem, add=True)
            return _

        lax.fori_loop(0, RPT, issue, None)

        def drain(r, _):
            pltpu.make_async_copy(z_h.at[pl.ds(0, BLK)], drow, sem).wait()
            return _

        lax.fori_loop(0, RPT, drain, None)
        plsc.subcore_barrier()
        pltpu.sync_copy(shared.at[pl.ds(s * NPS2, NPS2)],
                        out_h.at[c, pl.ds(s * NPS2, NPS2)])

    return k(src2d, w2d, z1)


def _tc_dis(deg_p):
    def body(deg_ref, dis_ref):
        d = deg_ref[0, :] + deg_ref[1, :]
        dis_ref[...] = jnp.where(d > 0.0, lax.rsqrt(d), 0.0)

    return pl.pallas_call(
        body,
        out_shape=jax.ShapeDtypeStruct((N2,), jnp.float32),
    )(deg_p)


def _norm_call(dis1d, src2d, dst2d, w2d):
    @functools.partial(
        pl.kernel,
        out_type=jax.ShapeDtypeStruct((R, BLK), jnp.float32),
        mesh=_mesh(),
        compiler_params=_sc_params(),
        scratch_types=[
            pltpu.VMEM((N2,), jnp.float32),
            pltpu.VMEM((RPT, BLK), jnp.int32),
            pltpu.VMEM((RPT, BLK), jnp.int32),
            pltpu.VMEM((RPT, BLK), jnp.float32),
            pltpu.VMEM((RPT, BLK), jnp.float32),
        ],
    )
    def k(dis_h, src_h, dst_h, w_h, out_h, dis_v, si_a, di_a, w_a, o_a):
        base = _wid() * RPT
        pltpu.sync_copy(dis_h, dis_v)
        pltpu.sync_copy(src_h.at[pl.ds(base, RPT)], si_a)
        pltpu.sync_copy(dst_h.at[pl.ds(base, RPT)], di_a)
        pltpu.sync_copy(w_h.at[pl.ds(base, RPT)], w_a)

        def body(r, _):
            for j in range(BLK // 16):
                sl = pl.ds(j * 16, 16)
                ds_ = plsc.load_gather(dis_v, [si_a[r, sl]])
                dd_ = plsc.load_gather(dis_v, [di_a[r, sl]])
                o_a[r, sl] = -(ds_ * w_a[r, sl] * dd_)
            return _

        lax.fori_loop(0, RPT, body, None)
        pltpu.sync_copy(o_a, out_h.at[pl.ds(base, RPT)])

    return k(dis1d, src2d, dst2d, w2d)


def _prop_call(table, src2d, dst2d, nn2d, zF, F):
    @functools.partial(
        pl.kernel,
        out_type=jax.ShapeDtypeStruct((NC, N2, F), jnp.float32),
        mesh=_mesh(),
        compiler_params=_sc_params(),
        scratch_types=[
            pltpu.VMEM_SHARED((N2, F), jnp.float32),
            pltpu.VMEM((RPT, BLK), jnp.int32),
            pltpu.VMEM((RPT, BLK), jnp.int32),
            pltpu.VMEM((RPT, BLK), jnp.float32),
            pltpu.VMEM((2, BLK, F), jnp.float32),
            pltpu.VMEM((2, BLK, F), jnp.float32),
            pltpu.SemaphoreType.DMA,
            pltpu.SemaphoreType.DMA,
            pltpu.SemaphoreType.DMA,
            pltpu.SemaphoreType.DMA,
        ],
    )
    def k(tab_h, src_h, dst_h, nn_h, z_h, out_h,
          shared, si_a, di_a, nn_a, gbuf, sbuf, sg0, sg1, ss0, ss1):
        c = lax.axis_index("c")
        s = lax.axis_index("s")
        base = _wid() * RPT
        pltpu.sync_copy(z_h.at[pl.ds(s * NPS2, NPS2)],
                        shared.at[pl.ds(s * NPS2, NPS2)])
        pltpu.sync_copy(src_h.at[pl.ds(base, RPT)], si_a)
        pltpu.sync_copy(dst_h.at[pl.ds(base, RPT)], di_a)
        pltpu.sync_copy(nn_h.at[pl.ds(base, RPT)], nn_a)
        plsc.subcore_barrier()

        sems_g = (sg0, sg1)
        sems_s = (ss0, ss1)

        def stage(r, cur, nxt):
            @pl.when(r + 1 < RPT)
            def _():
                pltpu.async_copy(tab_h.at[si_a.at[r + 1]], gbuf.at[nxt],
                                 sems_g[nxt])

            pltpu.make_async_copy(tab_h.at[pl.ds(0, BLK)], gbuf.at[cur],
                                  sems_g[cur]).wait()

            @pl.when(r >= 2)
            def _():
                pltpu.make_async_copy(z_h.at[pl.ds(0, BLK)], sbuf.at[cur],
                                      sems_s[cur]).wait()

            def scale(g, _):
                b16 = g * 16
                nnvec = nn_a[r, pl.ds(b16, 16)]
                for i in range(16):
                    sc = nnvec[i]
                    for j in range(F // 16):
                        sl = pl.ds(j * 16, 16)
                        sbuf[cur, b16 + i, sl] = gbuf[cur, b16 + i, sl] * sc
                return _

            lax.fori_loop(0, BLK // 16, scale, None)
            pltpu.async_copy(sbuf.at[cur], shared.at[di_a.at[r]],
                             sems_s[cur], add=True)

        pltpu.async_copy(tab_h.at[si_a.at[0]], gbuf.at[0], sg0)

        def body(kk, _):
            r = kk * 2
            stage(r, 0, 1)

            @pl.when(r + 1 < RPT)
            def _():
                stage(r + 1, 1, 0)

            return _

        lax.fori_loop(0, (RPT + 1) // 2, body, None)
        pltpu.make_async_copy(z_h.at[pl.ds(0, BLK)], sbuf.at[(RPT - 2) % 2],
                              sems_s[(RPT - 2) % 2]).wait()
        pltpu.make_async_copy(z_h.at[pl.ds(0, BLK)], sbuf.at[(RPT - 1) % 2],
                              sems_s[(RPT - 1) % 2]).wait()
        plsc.subcore_barrier()
        pltpu.sync_copy(shared.at[pl.ds(s * NPS2, NPS2)],
                        out_h.at[c, pl.ds(s * NPS2, NPS2)])

    return k(table, src2d, dst2d, nn2d, zF)


def _prop2_call(p1a, p1b, src2d, dst2d, nn2d, z16):
    @functools.partial(
        pl.kernel,
        out_type=jax.ShapeDtypeStruct((NC, N2, 16), jnp.float32),
        mesh=_mesh(),
        compiler_params=_sc_params(),
        scratch_types=[
            pltpu.VMEM_SHARED((N2, 16), jnp.float32),
            pltpu.VMEM((RPT, BLK), jnp.int32),
            pltpu.VMEM((RPT, BLK), jnp.int32),
            pltpu.VMEM((RPT, BLK), jnp.float32),
            pltpu.VMEM((2, BLK, 32), jnp.float32),
            pltpu.VMEM((2, BLK, 32), jnp.float32),
            pltpu.VMEM((2, BLK, 16), jnp.float32),
            pltpu.SemaphoreType.DMA,
            pltpu.SemaphoreType.DMA,
            pltpu.SemaphoreType.DMA,
            pltpu.SemaphoreType.DMA,
        ],
    )
    def k(pa_h, pb_h, src_h, dst_h, nn_h, z_h, out_h,
          shared, si_a, di_a, nn_a, ga, gb, sbuf, sg0, sg1, ss0, ss1):
        c = lax.axis_index("c")
        s = lax.axis_index("s")
        base = _wid() * RPT
        pltpu.sync_copy(z_h.at[pl.ds(s * NPS2, NPS2)],
                        shared.at[pl.ds(s * NPS2, NPS2)])
        pltpu.sync_copy(src_h.at[pl.ds(base, RPT)], si_a)
        pltpu.sync_copy(dst_h.at[pl.ds(base, RPT)], di_a)
        pltpu.sync_copy(nn_h.at[pl.ds(base, RPT)], nn_a)
        plsc.subcore_barrier()

        sems_g = (sg0, sg1)
        sems_s = (ss0, ss1)

        def issue_gathers(r, buf):
            pltpu.async_copy(pa_h.at[si_a.at[r]], ga.at[buf], sems_g[buf])
            pltpu.async_copy(pb_h.at[si_a.at[r]], gb.at[buf], sems_g[buf])

        def stage(r, cur, nxt):
            @pl.when(r + 1 < RPT)
            def _():
                issue_gathers(r + 1, nxt)

            pltpu.make_async_copy(pa_h.at[pl.ds(0, BLK)], ga.at[cur],
                                  sems_g[cur]).wait()
            pltpu.make_async_copy(pa_h.at[pl.ds(0, BLK)], gb.at[cur],
                                  sems_g[cur]).wait()

            @pl.when(r >= 2)
            def _():
                pltpu.make_async_copy(z_h.at[pl.ds(0, BLK)], sbuf.at[cur],
                                      sems_s[cur]).wait()

            def scale(g, _):
                b16 = g * 16
                nnvec = nn_a[r, pl.ds(b16, 16)]
                shi = pl.ds(16, 16)
                for i in range(16):
                    sc = nnvec[i]
                    sbuf[cur, b16 + i, :] = (ga[cur, b16 + i, shi]
                                             + gb[cur, b16 + i, shi]) * sc
                return _

            lax.fori_loop(0, BLK // 16, scale, None)
            pltpu.async_copy(sbuf.at[cur], shared.at[di_a.at[r]],
                             sems_s[cur], add=True)

        issue_gathers(0, 0)

        def body(kk, _):
            r = kk * 2
            stage(r, 0, 1)

            @pl.when(r + 1 < RPT)
            def _():
                stage(r + 1, 1, 0)

            return _

        lax.fori_loop(0, (RPT + 1) // 2, body, None)
        pltpu.make_async_copy(z_h.at[pl.ds(0, BLK)], sbuf.at[(RPT - 2) % 2],
                              sems_s[(RPT - 2) % 2]).wait()
        pltpu.make_async_copy(z_h.at[pl.ds(0, BLK)], sbuf.at[(RPT - 1) % 2],
                              sems_s[(RPT - 1) % 2]).wait()
        plsc.subcore_barrier()
        pltpu.sync_copy(shared.at[pl.ds(s * NPS2, NPS2)],
                        out_h.at[c, pl.ds(s * NPS2, NPS2)])

    return k(p1a, p1b, src2d, dst2d, nn2d, z16)


def _tc_in(x, Wcat):
    def body(x_ref, w_ref, zA_ref, zBC_ref):
        h = lax.dot_general(x_ref[...], w_ref[...], (((1,), (0,)), ((), ())),
                            precision=lax.Precision.HIGHEST,
                            preferred_element_type=jnp.float32)
        zA_ref[...] = h[:, :16]
        zBC_ref[...] = h[:, 16:48]

    BN = N2 // 8
    return pl.pallas_call(
        body,
        grid=(8,),
        in_specs=[pl.BlockSpec((BN, D), lambda i: (i, 0)),
                  pl.BlockSpec((D, 48), lambda i: (0, 0))],
        out_specs=[pl.BlockSpec((BN, 16), lambda i: (i, 0)),
                   pl.BlockSpec((BN, 32), lambda i: (i, 0))],
        out_shape=(jax.ShapeDtypeStruct((N2, 16), jnp.float32),
                   jax.ShapeDtypeStruct((N2, 32), jnp.float32)),
    )(x, Wcat)


def _tc_combine(zA, p1a, p1b, p2a, p2b, brow, Wcat):
    def body(zA_ref, p1a_ref, p1b_ref, p2a_ref, p2b_ref, b_ref, w_ref,
             zA2_ref, zBC2_ref, h_ref):
        act = (zA_ref[...] + p1a_ref[:, :16] + p1b_ref[:, :16]
               + 2.0 * (p2a_ref[...] + p2b_ref[...]) + b_ref[...])
        act = jnp.maximum(act, 0.0)
        h_ref[...] = act
        z = lax.dot_general(act, w_ref[...], (((1,), (0,)), ((), ())),
                            precision=lax.Precision.HIGHEST,
                            preferred_element_type=jnp.float32)
        zA2_ref[...] = z[:, :16]
        zBC2_ref[...] = z[:, 16:48]

    BN = N2 // 8
    return pl.pallas_call(
        body,
        grid=(8,),
        in_specs=[pl.BlockSpec((BN, 16), lambda i: (i, 0)),
                  pl.BlockSpec((BN, 32), lambda i: (i, 0)),
                  pl.BlockSpec((BN, 32), lambda i: (i, 0)),
                  pl.BlockSpec((BN, 16), lambda i: (i, 0)),
                  pl.BlockSpec((BN, 16), lambda i: (i, 0)),
                  pl.BlockSpec((1, 16), lambda i: (0, 0)),
                  pl.BlockSpec((16, 48), lambda i: (0, 0))],
        out_specs=[pl.BlockSpec((BN, 16), lambda i: (i, 0)),
                   pl.BlockSpec((BN, 32), lambda i: (i, 0)),
                   pl.BlockSpec((BN, 16), lambda i: (i, 0))],
        out_shape=(jax.ShapeDtypeStruct((N2, 16), jnp.float32),
                   jax.ShapeDtypeStruct((N2, 32), jnp.float32),
                   jax.ShapeDtypeStruct((N2, 16), jnp.float32)),
    )(zA, p1a, p1b, p2a, p2b, brow, Wcat)


def _tc_final(zA3, p1a, p1b, p2a, p2b, brow, h2, A2w, A2b):
    def body(zA_ref, p1a_ref, p1b_ref, p2a_ref, p2b_ref, b_ref, h2_ref,
             aw_ref, ab_ref, choice_ref, value_ref):
        cfull = (zA_ref[...] + p1a_ref[:, :16] + p1b_ref[:, :16]
                 + 2.0 * (p2a_ref[...] + p2b_ref[...]) + b_ref[...])
        valid = lax.broadcasted_iota(jnp.int32, (N2, 1), 0) < N
        c = jnp.where(valid, cfull[:, 0:1], -jnp.inf)
        m = jnp.max(c)
        ex = jnp.exp(c - m)
        choice_ref[...] = ex / jnp.sum(ex)
        v = jnp.sum(jnp.where(valid, h2_ref[...], 0.0), axis=0,
                    keepdims=True) * (1.0 / N)
        value_ref[...] = (
            jnp.sum(v * aw_ref[...], axis=1, keepdims=True) + ab_ref[...])

    return pl.pallas_call(
        body,
        out_shape=(jax.ShapeDtypeStruct((N2, 1), jnp.float32),
                   jax.ShapeDtypeStruct((1, 1), jnp.float32)),
    )(zA3, p1a, p1b, p2a, p2b, brow, h2, A2w, A2b)


def kernel(x, edge_index, weight, W1, b1, W2, b2, W3, b3, A2w, A2b):
    pad = EP - E
    src = jnp.pad(edge_index[0], (0, pad)).reshape(R, BLK)
    dst = jnp.pad(edge_index[1], (0, pad)).reshape(R, BLK)
    w2d = jnp.pad(weight, (0, pad)).reshape(R, BLK)

    W1cat = jnp.concatenate([W1[0] - W1[2], W1[1], W1[2]], axis=1)
    W2cat = jnp.concatenate([W2[0] - W2[2], W2[1], W2[2]], axis=1)
    W3p = jnp.pad(W3, ((0, 0), (0, 0), (0, 15)))
    W3cat = jnp.concatenate([W3p[0] - W3p[2], W3p[1], W3p[2]], axis=1)
    b1r = b1.reshape(1, 16)
    b2r = b2.reshape(1, 16)
    b3r = jnp.pad(b3, (0, 15)).reshape(1, 16)

    z1 = jnp.zeros((N2,), jnp.float32)
    z16 = jnp.zeros((N2, 16), jnp.float32)
    z32 = jnp.zeros((N2, 32), jnp.float32)

    deg_p = _deg_call(src, w2d, z1)
    dis1d = _tc_dis(deg_p)
    nn2d = _norm_call(dis1d, src, dst, w2d)

    xp = jnp.pad(x, ((0, N2 - N), (0, 0)))
    zA, zBC = _tc_in(xp, W1cat)
    for layer in range(3):
        p1 = _prop_call(zBC, src, dst, nn2d, z32, 32)
        p1a, p1b = p1[0], p1[1]
        p2 = _prop2_call(p1a, p1b, src, dst, nn2d, z16)
        if layer == 0:
            zA, zBC, _ = _tc_combine(zA, p1a, p1b, p2[0], p2[1], b1r, W2cat)
        elif layer == 1:
            zA, zBC, h2 = _tc_combine(zA, p1a, p1b, p2[0], p2[1], b2r, W3cat)
        else:
            choice, value = _tc_final(zA, p1a, p1b, p2[0], p2[1], b3r, h2,
                                      A2w, A2b.reshape(1, 1))
    return choice[:N, 0], value.reshape(())


# speedup vs baseline: 16.7133x; 1.0321x over previous
import functools

import jax
import jax.numpy as jnp
from jax import lax
from jax.experimental import pallas as pl
from jax.experimental.pallas import tpu as pltpu
from jax.experimental.pallas import tpu_sc as plsc

N = 10000
E = 320000
D = 128
H = 16

NC = 2
NS = 16
NW = NC * NS
BLK = 128
EP = ((E + NW * BLK - 1) // (NW * BLK)) * (NW * BLK)
R = EP // BLK
RPT = R // NW
N2 = 10240
NPS2 = N2 // NS

_mesh = functools.partial(
    plsc.VectorSubcoreMesh, core_axis_name="c", subcore_axis_name="s")

_sc_params = functools.partial(
    pltpu.CompilerParams, needs_layout_passes=False, use_tc_tiling_on_sc=False)


def _wid():
    return lax.axis_index("c") * NS + lax.axis_index("s")


def _deg_call(src2d, w2d, z1):
    @functools.partial(
        pl.kernel,
        out_type=jax.ShapeDtypeStruct((NC, N2), jnp.float32),
        mesh=_mesh(),
        compiler_params=_sc_params(),
        scratch_types=[
            pltpu.VMEM_SHARED((N2,), jnp.float32),
            pltpu.VMEM((RPT, BLK), jnp.int32),
            pltpu.VMEM((RPT, BLK), jnp.float32),
            pltpu.VMEM((BLK,), jnp.float32),
            pltpu.SemaphoreType.DMA,
        ],
    )
    def k(src_h, w_h, z_h, out_h, shared, si_a, w_a, drow, sem):
        c = lax.axis_index("c")
        s = lax.axis_index("s")
        base = _wid() * RPT
        pltpu.sync_copy(z_h.at[pl.ds(s * NPS2, NPS2)],
                        shared.at[pl.ds(s * NPS2, NPS2)])
        pltpu.sync_copy(src_h.at[pl.ds(base, RPT)], si_a)
        pltpu.sync_copy(w_h.at[pl.ds(base, RPT)], w_a)
        plsc.subcore_barrier()

        def issue(r, _):
            pltpu.async_copy(w_a.at[r], shared.at[si_a.at[r]], sem, add=True)
            return _

        lax.fori_loop(0, RPT, issue, None)

        def drain(r, _):
            pltpu.make_async_copy(z_h.at[pl.ds(0, BLK)], drow, sem).wait()
            return _

        lax.fori_loop(0, RPT, drain, None)
        plsc.subcore_barrier()
        pltpu.sync_copy(shared.at[pl.ds(s * NPS2, NPS2)],
                        out_h.at[c, pl.ds(s * NPS2, NPS2)])

    return k(src2d, w2d, z1)


def _prop_call(table, src2d, dst2d, nn2d, zF, F, dis_w=None):
    fuse_norm = dis_w is not None
    out_type = jax.ShapeDtypeStruct((NC, N2, F), jnp.float32)
    if fuse_norm:
        out_type = (out_type, jax.ShapeDtypeStruct((R, BLK), jnp.float32))

    @functools.partial(
        pl.kernel,
        out_type=out_type,
        mesh=_mesh(),
        compiler_params=_sc_params(),
        scratch_types=[
            pltpu.VMEM_SHARED((N2, F), jnp.float32),
            pltpu.VMEM((RPT, BLK), jnp.int32),
            pltpu.VMEM((RPT, BLK), jnp.int32),
            pltpu.VMEM((RPT, BLK), jnp.float32),
            pltpu.VMEM((2, BLK, F), jnp.float32),
            pltpu.VMEM((2, BLK, F), jnp.float32),
            pltpu.SemaphoreType.DMA,
            pltpu.SemaphoreType.DMA,
            pltpu.SemaphoreType.DMA,
            pltpu.SemaphoreType.DMA,
        ] + ([pltpu.VMEM((N2,), jnp.float32),
              pltpu.VMEM((RPT, BLK), jnp.float32)] if fuse_norm else []),
    )
    def k(tab_h, src_h, dst_h, *rest):
        if fuse_norm:
            (dis_h, w_h, z_h, out_h, nn_out_h,
             shared, si_a, di_a, nn_a, gbuf, sbuf,
             sg0, sg1, ss0, ss1, dis_v, w_a) = rest
        else:
            (nn_h, z_h, out_h,
             shared, si_a, di_a, nn_a, gbuf, sbuf,
             sg0, sg1, ss0, ss1) = rest
        c = lax.axis_index("c")
        s = lax.axis_index("s")
        base = _wid() * RPT
        pltpu.sync_copy(z_h.at[pl.ds(s * NPS2, NPS2)],
                        shared.at[pl.ds(s * NPS2, NPS2)])
        pltpu.sync_copy(src_h.at[pl.ds(base, RPT)], si_a)
        pltpu.sync_copy(dst_h.at[pl.ds(base, RPT)], di_a)
        if fuse_norm:
            pltpu.sync_copy(dis_h, dis_v)
            pltpu.sync_copy(w_h.at[pl.ds(base, RPT)], w_a)

            def mk_nn(r, _):
                for j in range(BLK // 16):
                    sl = pl.ds(j * 16, 16)
                    ds_ = plsc.load_gather(dis_v, [si_a[r, sl]])
                    dd_ = plsc.load_gather(dis_v, [di_a[r, sl]])
                    nn_a[r, sl] = -(ds_ * w_a[r, sl] * dd_)
                return _

            lax.fori_loop(0, RPT, mk_nn, None)
            pltpu.sync_copy(nn_a, nn_out_h.at[pl.ds(base, RPT)])
        else:
            pltpu.sync_copy(nn_h.at[pl.ds(base, RPT)], nn_a)
        plsc.subcore_barrier()

        sems_g = (sg0, sg1)
        sems_s = (ss0, ss1)

        def stage(r, cur, nxt):
            @pl.when(r + 1 < RPT)
            def _():
                pltpu.async_copy(tab_h.at[si_a.at[r + 1]], gbuf.at[nxt],
                                 sems_g[nxt])

            pltpu.make_async_copy(tab_h.at[pl.ds(0, BLK)], gbuf.at[cur],
                                  sems_g[cur]).wait()

            @pl.when(r >= 2)
            def _():
                pltpu.make_async_copy(z_h.at[pl.ds(0, BLK)], sbuf.at[cur],
                                      sems_s[cur]).wait()

            def scale(g, _):
                b16 = g * 16
                nnvec = nn_a[r, pl.ds(b16, 16)]
                for i in range(16):
                    sc = nnvec[i]
                    for j in range(F // 16):
                        sl = pl.ds(j * 16, 16)
                        sbuf[cur, b16 + i, sl] = gbuf[cur, b16 + i, sl] * sc
                return _

            lax.fori_loop(0, BLK // 16, scale, None)
            pltpu.async_copy(sbuf.at[cur], shared.at[di_a.at[r]],
                             sems_s[cur], add=True)

        pltpu.async_copy(tab_h.at[si_a.at[0]], gbuf.at[0], sg0)

        def body(kk, _):
            r = kk * 2
            stage(r, 0, 1)

            @pl.when(r + 1 < RPT)
            def _():
                stage(r + 1, 1, 0)

            return _

        lax.fori_loop(0, (RPT + 1) // 2, body, None)
        pltpu.make_async_copy(z_h.at[pl.ds(0, BLK)], sbuf.at[(RPT - 2) % 2],
                              sems_s[(RPT - 2) % 2]).wait()
        pltpu.make_async_copy(z_h.at[pl.ds(0, BLK)], sbuf.at[(RPT - 1) % 2],
                              sems_s[(RPT - 1) % 2]).wait()
        plsc.subcore_barrier()
        pltpu.sync_copy(shared.at[pl.ds(s * NPS2, NPS2)],
                        out_h.at[c, pl.ds(s * NPS2, NPS2)])

    if fuse_norm:
        return k(table, src2d, dst2d, dis_w[0], dis_w[1], zF)
    return k(table, src2d, dst2d, nn2d, zF)


def _prop2_call(p1a, p1b, src2d, dst2d, nn2d, z16, FW):
    @functools.partial(
        pl.kernel,
        out_type=jax.ShapeDtypeStruct((NC, N2, 16), jnp.float32),
        mesh=_mesh(),
        compiler_params=_sc_params(),
        scratch_types=[
            pltpu.VMEM_SHARED((N2, 16), jnp.float32),
            pltpu.VMEM((RPT, BLK), jnp.int32),
            pltpu.VMEM((RPT, BLK), jnp.int32),
            pltpu.VMEM((RPT, BLK), jnp.float32),
            pltpu.VMEM((2, BLK, FW), jnp.float32),
            pltpu.VMEM((2, BLK, FW), jnp.float32),
            pltpu.VMEM((2, BLK, 16), jnp.float32),
            pltpu.SemaphoreType.DMA,
            pltpu.SemaphoreType.DMA,
            pltpu.SemaphoreType.DMA,
            pltpu.SemaphoreType.DMA,
        ],
    )
    def k(pa_h, pb_h, src_h, dst_h, nn_h, z_h, out_h,
          shared, si_a, di_a, nn_a, ga, gb, sbuf, sg0, sg1, ss0, ss1):
        c = lax.axis_index("c")
        s = lax.axis_index("s")
        base = _wid() * RPT
        pltpu.sync_copy(z_h.at[pl.ds(s * NPS2, NPS2)],
                        shared.at[pl.ds(s * NPS2, NPS2)])
        pltpu.sync_copy(src_h.at[pl.ds(base, RPT)], si_a)
        pltpu.sync_copy(dst_h.at[pl.ds(base, RPT)], di_a)
        pltpu.sync_copy(nn_h.at[pl.ds(base, RPT)], nn_a)
        plsc.subcore_barrier()

        sems_g = (sg0, sg1)
        sems_s = (ss0, ss1)

        def issue_gathers(r, buf):
            pltpu.async_copy(pa_h.at[si_a.at[r]], ga.at[buf], sems_g[buf])
            pltpu.async_copy(pb_h.at[si_a.at[r]], gb.at[buf], sems_g[buf])

        def stage(r, cur, nxt):
            @pl.when(r + 1 < RPT)
            def _():
                issue_gathers(r + 1, nxt)

            pltpu.make_async_copy(pa_h.at[pl.ds(0, BLK)], ga.at[cur],
                                  sems_g[cur]).wait()
            pltpu.make_async_copy(pa_h.at[pl.ds(0, BLK)], gb.at[cur],
                                  sems_g[cur]).wait()

            @pl.when(r >= 2)
            def _():
                pltpu.make_async_copy(z_h.at[pl.ds(0, BLK)], sbuf.at[cur],
                                      sems_s[cur]).wait()

            def scale(g, _):
                b16 = g * 16
                nnvec = nn_a[r, pl.ds(b16, 16)]
                shi = pl.ds(FW - 16, 16)
                for i in range(16):
                    sc = nnvec[i]
                    sbuf[cur, b16 + i, :] = (ga[cur, b16 + i, shi]
                                             + gb[cur, b16 + i, shi]) * sc
                return _

            lax.fori_loop(0, BLK // 16, scale, None)
            pltpu.async_copy(sbuf.at[cur], shared.at[di_a.at[r]],
                             sems_s[cur], add=True)

        issue_gathers(0, 0)

        def body(kk, _):
            r = kk * 2
            stage(r, 0, 1)

            @pl.when(r + 1 < RPT)
            def _():
                stage(r + 1, 1, 0)

            return _

        lax.fori_loop(0, (RPT + 1) // 2, body, None)
        pltpu.make_async_copy(z_h.at[pl.ds(0, BLK)], sbuf.at[(RPT - 2) % 2],
                              sems_s[(RPT - 2) % 2]).wait()
        pltpu.make_async_copy(z_h.at[pl.ds(0, BLK)], sbuf.at[(RPT - 1) % 2],
                              sems_s[(RPT - 1) % 2]).wait()
        plsc.subcore_barrier()
        pltpu.sync_copy(shared.at[pl.ds(s * NPS2, NPS2)],
                        out_h.at[c, pl.ds(s * NPS2, NPS2)])

    return k(p1a, p1b, src2d, dst2d, nn2d, z16)


def _tc_in(x, Wcat, deg_p):
    def body(x_ref, w_ref, deg_ref, zA_ref, zBC_ref, dis_ref):
        h = lax.dot_general(x_ref[...], w_ref[...], (((1,), (0,)), ((), ())),
                            precision=lax.Precision.HIGHEST,
                            preferred_element_type=jnp.float32)
        zA_ref[...] = h[:, :16]
        zBC_ref[...] = h[:, 16:48]

        @pl.when(pl.program_id(0) == 0)
        def _():
            d = deg_ref[0, :] + deg_ref[1, :]
            dis_ref[...] = jnp.where(d > 0.0, lax.rsqrt(d), 0.0)

    BN = N2 // 8
    return pl.pallas_call(
        body,
        grid=(8,),
        in_specs=[pl.BlockSpec((BN, D), lambda i: (i, 0)),
                  pl.BlockSpec((D, 48), lambda i: (0, 0)),
                  pl.BlockSpec((NC, N2), lambda i: (0, 0))],
        out_specs=[pl.BlockSpec((BN, 16), lambda i: (i, 0)),
                   pl.BlockSpec((BN, 32), lambda i: (i, 0)),
                   pl.BlockSpec((N2,), lambda i: (0,))],
        out_shape=(jax.ShapeDtypeStruct((N2, 16), jnp.float32),
                   jax.ShapeDtypeStruct((N2, 32), jnp.float32),
                   jax.ShapeDtypeStruct((N2,), jnp.float32)),
    )(x, Wcat, deg_p)


def _tc_combine(zA, p1a, p1b, p2a, p2b, brow, Wcat, wbc):
    wz = 16 + wbc

    def body(zA_ref, p1a_ref, p1b_ref, p2a_ref, p2b_ref, b_ref, w_ref,
             zA2_ref, zBC2_ref, h_ref):
        act = (zA_ref[...] + p1a_ref[:, :16] + p1b_ref[:, :16]
               + 2.0 * (p2a_ref[...] + p2b_ref[...]) + b_ref[...])
        act = jnp.maximum(act, 0.0)
        h_ref[...] = act
        z = lax.dot_general(act, w_ref[...], (((1,), (0,)), ((), ())),
                            precision=lax.Precision.HIGHEST,
                            preferred_element_type=jnp.float32)
        zA2_ref[...] = z[:, :16]
        zBC2_ref[...] = z[:, 16:]

    BN = N2 // 8
    return pl.pallas_call(
        body,
        grid=(8,),
        in_specs=[pl.BlockSpec((BN, 16), lambda i: (i, 0)),
                  pl.BlockSpec((BN, 32), lambda i: (i, 0)),
                  pl.BlockSpec((BN, 32), lambda i: (i, 0)),
                  pl.BlockSpec((BN, 16), lambda i: (i, 0)),
                  pl.BlockSpec((BN, 16), lambda i: (i, 0)),
                  pl.BlockSpec((1, 16), lambda i: (0, 0)),
                  pl.BlockSpec((16, wz), lambda i: (0, 0))],
        out_specs=[pl.BlockSpec((BN, 16), lambda i: (i, 0)),
                   pl.BlockSpec((BN, wbc), lambda i: (i, 0)),
                   pl.BlockSpec((BN, 16), lambda i: (i, 0))],
        out_shape=(jax.ShapeDtypeStruct((N2, 16), jnp.float32),
                   jax.ShapeDtypeStruct((N2, wbc), jnp.float32),
                   jax.ShapeDtypeStruct((N2, 16), jnp.float32)),
    )(zA, p1a, p1b, p2a, p2b, brow, Wcat)


def _tc_final(zA3, p1a, p1b, p2a, p2b, brow, h2, A2w, A2b):
    def body(zA_ref, p1a_ref, p1b_ref, p2a_ref, p2b_ref, b_ref, h2_ref,
             aw_ref, ab_ref, choice_ref, value_ref):
        c0 = (zA_ref[:, 0:1] + p1a_ref[:, 0:1] + p1b_ref[:, 0:1]
              + 2.0 * (p2a_ref[:, 8:9] + p2b_ref[:, 8:9]) + b_ref[0, 0])
        valid = lax.broadcasted_iota(jnp.int32, (N2, 1), 0) < N
        c = jnp.where(valid, c0, -jnp.inf)
        m = jnp.max(c)
        ex = jnp.exp(c - m)
        choice_ref[...] = ex / jnp.sum(ex)
        v = jnp.sum(jnp.where(valid, h2_ref[...], 0.0), axis=0,
                    keepdims=True) * (1.0 / N)
        value_ref[...] = (
            jnp.sum(v * aw_ref[...], axis=1, keepdims=True) + ab_ref[...])

    return pl.pallas_call(
        body,
        out_shape=(jax.ShapeDtypeStruct((N2, 1), jnp.float32),
                   jax.ShapeDtypeStruct((1, 1), jnp.float32)),
    )(zA3, p1a, p1b, p2a, p2b, brow, h2, A2w, A2b)


def kernel(x, edge_index, weight, W1, b1, W2, b2, W3, b3, A2w, A2b):
    pad = EP - E
    src = jnp.pad(edge_index[0], (0, pad)).reshape(R, BLK)
    dst = jnp.pad(edge_index[1], (0, pad)).reshape(R, BLK)
    w2d = jnp.pad(weight, (0, pad)).reshape(R, BLK)

    W1cat = jnp.concatenate([W1[0] - W1[2], W1[1], W1[2]], axis=1)
    W2cat = jnp.concatenate([W2[0] - W2[2], W2[1], W2[2]], axis=1)
    W3p16 = jnp.pad(W3, ((0, 0), (0, 0), (0, 15)))
    W3p8 = jnp.pad(W3, ((0, 0), (0, 0), (0, 7)))
    W3cat = jnp.concatenate([W3p16[0] - W3p16[2], W3p8[1], W3p8[2]], axis=1)
    b1r = b1.reshape(1, 16)
    b2r = b2.reshape(1, 16)
    b3r = jnp.pad(b3, (0, 15)).reshape(1, 16)

    z1 = jnp.zeros((N2,), jnp.float32)
    z16 = jnp.zeros((N2, 16), jnp.float32)
    z32 = jnp.zeros((N2, 32), jnp.float32)

    deg_p = _deg_call(src, w2d, z1)
    xp = jnp.pad(x, ((0, N2 - N), (0, 0)))
    zA, zBC, dis1d = _tc_in(xp, W1cat, deg_p)

    p1, nn2d = _prop_call(zBC, src, dst, None, z32, 32, dis_w=(dis1d, w2d))
    p2 = _prop2_call(p1[0], p1[1], src, dst, nn2d, z16, 32)
    zA, zBC, _ = _tc_combine(zA, p1[0], p1[1], p2[0], p2[1], b1r, W2cat, 32)

    p1 = _prop_call(zBC, src, dst, nn2d, z32, 32)
    p2 = _prop2_call(p1[0], p1[1], src, dst, nn2d, z16, 32)
    zA, zBC, h2 = _tc_combine(zA, p1[0], p1[1], p2[0], p2[1], b2r, W3cat, 16)

    p1 = _prop_call(zBC, src, dst, nn2d, z16, 16)
    p2 = _prop2_call(p1[0], p1[1], src, dst, nn2d, z16, 16)
    choice, value = _tc_final(zA, p1[0], p1[1], p2[0], p2[1], b3r, h2,
                              A2w, A2b.reshape(1, 1))
    return choice[:N, 0], value.reshape(())


# speedup vs baseline: 17.1023x; 1.0233x over previous
import functools

import jax
import jax.numpy as jnp
from jax import lax
from jax.experimental import pallas as pl
from jax.experimental.pallas import tpu as pltpu
from jax.experimental.pallas import tpu_sc as plsc

N = 10000
E = 320000
D = 128
H = 16

NC = 2
NS = 16
NW = NC * NS
BLK = 128
EP = ((E + NW * BLK - 1) // (NW * BLK)) * (NW * BLK)
R = EP // BLK
RPT = R // NW
N2 = 10240
NPS2 = N2 // NS

_mesh = functools.partial(
    plsc.VectorSubcoreMesh, core_axis_name="c", subcore_axis_name="s")

_sc_params = functools.partial(
    pltpu.CompilerParams, needs_layout_passes=False, use_tc_tiling_on_sc=False)


def _wid():
    return lax.axis_index("c") * NS + lax.axis_index("s")


def _deg_call(src2d, w2d, z1):
    @functools.partial(
        pl.kernel,
        out_type=jax.ShapeDtypeStruct((NC, N2), jnp.float32),
        mesh=_mesh(),
        compiler_params=_sc_params(),
        scratch_types=[
            pltpu.VMEM_SHARED((N2,), jnp.float32),
            pltpu.VMEM((RPT, BLK), jnp.int32),
            pltpu.VMEM((RPT, BLK), jnp.float32),
            pltpu.VMEM((BLK,), jnp.float32),
            pltpu.SemaphoreType.DMA,
        ],
    )
    def k(src_h, w_h, z_h, out_h, shared, si_a, w_a, drow, sem):
        c = lax.axis_index("c")
        s = lax.axis_index("s")
        base = _wid() * RPT
        pltpu.sync_copy(z_h.at[pl.ds(s * NPS2, NPS2)],
                        shared.at[pl.ds(s * NPS2, NPS2)])
        pltpu.sync_copy(src_h.at[pl.ds(base, RPT)], si_a)
        pltpu.sync_copy(w_h.at[pl.ds(base, RPT)], w_a)
        plsc.subcore_barrier()

        def issue(r, _):
            pltpu.async_copy(w_a.at[r], shared.at[si_a.at[r]], sem, add=True)
            return _

        lax.fori_loop(0, RPT, issue, None)

        def drain(r, _):
            pltpu.make_async_copy(z_h.at[pl.ds(0, BLK)], drow, sem).wait()
            return _

        lax.fori_loop(0, RPT, drain, None)
        plsc.subcore_barrier()
        pltpu.sync_copy(shared.at[pl.ds(s * NPS2, NPS2)],
                        out_h.at[c, pl.ds(s * NPS2, NPS2)])

    return k(src2d, w2d, z1)


def _prop_call(table, src2d, dst2d, nn2d, zF, F, dis_w=None):
    fuse_norm = dis_w is not None
    out_type = jax.ShapeDtypeStruct((NC, N2, F), jnp.float32)
    if fuse_norm:
        out_type = (out_type, jax.ShapeDtypeStruct((R, BLK), jnp.float32))

    @functools.partial(
        pl.kernel,
        out_type=out_type,
        mesh=_mesh(),
        compiler_params=_sc_params(),
        scratch_types=[
            pltpu.VMEM_SHARED((N2, F), jnp.float32),
            pltpu.VMEM((RPT, BLK), jnp.int32),
            pltpu.VMEM((RPT, BLK), jnp.int32),
            pltpu.VMEM((RPT, BLK), jnp.float32),
            pltpu.VMEM((2, BLK, F), jnp.float32),
            pltpu.VMEM((2, BLK, F), jnp.float32),
            pltpu.SemaphoreType.DMA,
            pltpu.SemaphoreType.DMA,
            pltpu.SemaphoreType.DMA,
            pltpu.SemaphoreType.DMA,
        ] + ([pltpu.VMEM((N2,), jnp.float32),
              pltpu.VMEM((RPT, BLK), jnp.float32)] if fuse_norm else []),
    )
    def k(tab_h, src_h, dst_h, *rest):
        if fuse_norm:
            (dis_h, w_h, z_h, out_h, nn_out_h,
             shared, si_a, di_a, nn_a, gbuf, sbuf,
             sg0, sg1, ss0, ss1, dis_v, w_a) = rest
        else:
            (nn_h, z_h, out_h,
             shared, si_a, di_a, nn_a, gbuf, sbuf,
             sg0, sg1, ss0, ss1) = rest
        c = lax.axis_index("c")
        s = lax.axis_index("s")
        base = _wid() * RPT
        pltpu.sync_copy(z_h.at[pl.ds(s * NPS2, NPS2)],
                        shared.at[pl.ds(s * NPS2, NPS2)])
        pltpu.sync_copy(src_h.at[pl.ds(base, RPT)], si_a)
        pltpu.sync_copy(dst_h.at[pl.ds(base, RPT)], di_a)
        if fuse_norm:
            pltpu.sync_copy(dis_h, dis_v)
            pltpu.sync_copy(w_h.at[pl.ds(base, RPT)], w_a)

            def mk_nn(r, _):
                for j in range(BLK // 16):
                    sl = pl.ds(j * 16, 16)
                    ds_ = plsc.load_gather(dis_v, [si_a[r, sl]])
                    dd_ = plsc.load_gather(dis_v, [di_a[r, sl]])
                    nn_a[r, sl] = -(ds_ * w_a[r, sl] * dd_)
                return _

            lax.fori_loop(0, RPT, mk_nn, None)
            pltpu.sync_copy(nn_a, nn_out_h.at[pl.ds(base, RPT)])
        else:
            pltpu.sync_copy(nn_h.at[pl.ds(base, RPT)], nn_a)
        plsc.subcore_barrier()

        sems_g = (sg0, sg1)
        sems_s = (ss0, ss1)

        def stage(r, cur, nxt):
            @pl.when(r + 1 < RPT)
            def _():
                pltpu.async_copy(tab_h.at[si_a.at[r + 1]], gbuf.at[nxt],
                                 sems_g[nxt])

            pltpu.make_async_copy(tab_h.at[pl.ds(0, BLK)], gbuf.at[cur],
                                  sems_g[cur]).wait()

            @pl.when(r >= 2)
            def _():
                pltpu.make_async_copy(z_h.at[pl.ds(0, BLK)], sbuf.at[cur],
                                      sems_s[cur]).wait()

            for g in range(BLK // 16):
                b16 = g * 16
                nnvec = nn_a[r, pl.ds(b16, 16)]
                for i in range(16):
                    sc = nnvec[i]
                    for j in range(F // 16):
                        sl = pl.ds(j * 16, 16)
                        sbuf[cur, b16 + i, sl] = gbuf[cur, b16 + i, sl] * sc
            pltpu.async_copy(sbuf.at[cur], shared.at[di_a.at[r]],
                             sems_s[cur], add=True)

        pltpu.async_copy(tab_h.at[si_a.at[0]], gbuf.at[0], sg0)

        def body(kk, _):
            r = kk * 2
            stage(r, 0, 1)

            @pl.when(r + 1 < RPT)
            def _():
                stage(r + 1, 1, 0)

            return _

        lax.fori_loop(0, (RPT + 1) // 2, body, None)
        pltpu.make_async_copy(z_h.at[pl.ds(0, BLK)], sbuf.at[(RPT - 2) % 2],
                              sems_s[(RPT - 2) % 2]).wait()
        pltpu.make_async_copy(z_h.at[pl.ds(0, BLK)], sbuf.at[(RPT - 1) % 2],
                              sems_s[(RPT - 1) % 2]).wait()
        plsc.subcore_barrier()
        pltpu.sync_copy(shared.at[pl.ds(s * NPS2, NPS2)],
                        out_h.at[c, pl.ds(s * NPS2, NPS2)])

    if fuse_norm:
        return k(table, src2d, dst2d, dis_w[0], dis_w[1], zF)
    return k(table, src2d, dst2d, nn2d, zF)


def _prop2_call(p1a, p1b, src2d, dst2d, nn2d, z16, FW):
    @functools.partial(
        pl.kernel,
        out_type=jax.ShapeDtypeStruct((NC, N2, 16), jnp.float32),
        mesh=_mesh(),
        compiler_params=_sc_params(),
        scratch_types=[
            pltpu.VMEM_SHARED((N2, 16), jnp.float32),
            pltpu.VMEM((RPT, BLK), jnp.int32),
            pltpu.VMEM((RPT, BLK), jnp.int32),
            pltpu.VMEM((RPT, BLK), jnp.float32),
            pltpu.VMEM((2, BLK, FW), jnp.float32),
            pltpu.VMEM((2, BLK, FW), jnp.float32),
            pltpu.VMEM((2, BLK, 16), jnp.float32),
            pltpu.SemaphoreType.DMA,
            pltpu.SemaphoreType.DMA,
            pltpu.SemaphoreType.DMA,
            pltpu.SemaphoreType.DMA,
        ],
    )
    def k(pa_h, pb_h, src_h, dst_h, nn_h, z_h, out_h,
          shared, si_a, di_a, nn_a, ga, gb, sbuf, sg0, sg1, ss0, ss1):
        c = lax.axis_index("c")
        s = lax.axis_index("s")
        base = _wid() * RPT
        pltpu.sync_copy(z_h.at[pl.ds(s * NPS2, NPS2)],
                        shared.at[pl.ds(s * NPS2, NPS2)])
        pltpu.sync_copy(src_h.at[pl.ds(base, RPT)], si_a)
        pltpu.sync_copy(dst_h.at[pl.ds(base, RPT)], di_a)
        pltpu.sync_copy(nn_h.at[pl.ds(base, RPT)], nn_a)
        plsc.subcore_barrier()

        sems_g = (sg0, sg1)
        sems_s = (ss0, ss1)

        def issue_gathers(r, buf):
            pltpu.async_copy(pa_h.at[si_a.at[r]], ga.at[buf], sems_g[buf])
            pltpu.async_copy(pb_h.at[si_a.at[r]], gb.at[buf], sems_g[buf])

        def stage(r, cur, nxt):
            @pl.when(r + 1 < RPT)
            def _():
                issue_gathers(r + 1, nxt)

            pltpu.make_async_copy(pa_h.at[pl.ds(0, BLK)], ga.at[cur],
                                  sems_g[cur]).wait()
            pltpu.make_async_copy(pa_h.at[pl.ds(0, BLK)], gb.at[cur],
                                  sems_g[cur]).wait()

            @pl.when(r >= 2)
            def _():
                pltpu.make_async_copy(z_h.at[pl.ds(0, BLK)], sbuf.at[cur],
                                      sems_s[cur]).wait()

            for g in range(BLK // 16):
                b16 = g * 16
                nnvec = nn_a[r, pl.ds(b16, 16)]
                shi = pl.ds(FW - 16, 16)
                for i in range(16):
                    sc = nnvec[i]
                    sbuf[cur, b16 + i, :] = (ga[cur, b16 + i, shi]
                                             + gb[cur, b16 + i, shi]) * sc
            pltpu.async_copy(sbuf.at[cur], shared.at[di_a.at[r]],
                             sems_s[cur], add=True)

        issue_gathers(0, 0)

        def body(kk, _):
            r = kk * 2
            stage(r, 0, 1)

            @pl.when(r + 1 < RPT)
            def _():
                stage(r + 1, 1, 0)

            return _

        lax.fori_loop(0, (RPT + 1) // 2, body, None)
        pltpu.make_async_copy(z_h.at[pl.ds(0, BLK)], sbuf.at[(RPT - 2) % 2],
                              sems_s[(RPT - 2) % 2]).wait()
        pltpu.make_async_copy(z_h.at[pl.ds(0, BLK)], sbuf.at[(RPT - 1) % 2],
                              sems_s[(RPT - 1) % 2]).wait()
        plsc.subcore_barrier()
        pltpu.sync_copy(shared.at[pl.ds(s * NPS2, NPS2)],
                        out_h.at[c, pl.ds(s * NPS2, NPS2)])

    return k(p1a, p1b, src2d, dst2d, nn2d, z16)


def _tc_in(x, Wcat, deg_p):
    def body(x_ref, w_ref, deg_ref, zA_ref, zBC_ref, dis_ref):
        h = lax.dot_general(x_ref[...], w_ref[...], (((1,), (0,)), ((), ())),
                            precision=lax.Precision.HIGHEST,
                            preferred_element_type=jnp.float32)
        zA_ref[...] = h[:, :16]
        zBC_ref[...] = h[:, 16:48]

        @pl.when(pl.program_id(0) == 0)
        def _():
            d = deg_ref[0, :] + deg_ref[1, :]
            dis_ref[...] = jnp.where(d > 0.0, lax.rsqrt(d), 0.0)

    BN = N2 // 8
    return pl.pallas_call(
        body,
        grid=(8,),
        in_specs=[pl.BlockSpec((BN, D), lambda i: (i, 0)),
                  pl.BlockSpec((D, 48), lambda i: (0, 0)),
                  pl.BlockSpec((NC, N2), lambda i: (0, 0))],
        out_specs=[pl.BlockSpec((BN, 16), lambda i: (i, 0)),
                   pl.BlockSpec((BN, 32), lambda i: (i, 0)),
                   pl.BlockSpec((N2,), lambda i: (0,))],
        out_shape=(jax.ShapeDtypeStruct((N2, 16), jnp.float32),
                   jax.ShapeDtypeStruct((N2, 32), jnp.float32),
                   jax.ShapeDtypeStruct((N2,), jnp.float32)),
    )(x, Wcat, deg_p)


def _tc_combine(zA, p1a, p1b, p2a, p2b, brow, Wcat, wbc):
    wz = 16 + wbc

    def body(zA_ref, p1a_ref, p1b_ref, p2a_ref, p2b_ref, b_ref, w_ref,
             zA2_ref, zBC2_ref, h_ref):
        act = (zA_ref[...] + p1a_ref[:, :16] + p1b_ref[:, :16]
               + 2.0 * (p2a_ref[...] + p2b_ref[...]) + b_ref[...])
        act = jnp.maximum(act, 0.0)
        h_ref[...] = act
        z = lax.dot_general(act, w_ref[...], (((1,), (0,)), ((), ())),
                            precision=lax.Precision.HIGHEST,
                            preferred_element_type=jnp.float32)
        zA2_ref[...] = z[:, :16]
        zBC2_ref[...] = z[:, 16:]

    BN = N2 // 8
    return pl.pallas_call(
        body,
        grid=(8,),
        in_specs=[pl.BlockSpec((BN, 16), lambda i: (i, 0)),
                  pl.BlockSpec((BN, 32), lambda i: (i, 0)),
                  pl.BlockSpec((BN, 32), lambda i: (i, 0)),
                  pl.BlockSpec((BN, 16), lambda i: (i, 0)),
                  pl.BlockSpec((BN, 16), lambda i: (i, 0)),
                  pl.BlockSpec((1, 16), lambda i: (0, 0)),
                  pl.BlockSpec((16, wz), lambda i: (0, 0))],
        out_specs=[pl.BlockSpec((BN, 16), lambda i: (i, 0)),
                   pl.BlockSpec((BN, wbc), lambda i: (i, 0)),
                   pl.BlockSpec((BN, 16), lambda i: (i, 0))],
        out_shape=(jax.ShapeDtypeStruct((N2, 16), jnp.float32),
                   jax.ShapeDtypeStruct((N2, wbc), jnp.float32),
                   jax.ShapeDtypeStruct((N2, 16), jnp.float32)),
    )(zA, p1a, p1b, p2a, p2b, brow, Wcat)


def _tc_final(zA3, p1a, p1b, p2a, p2b, brow, h2, A2w, A2b):
    def body(zA_ref, p1a_ref, p1b_ref, p2a_ref, p2b_ref, b_ref, h2_ref,
             aw_ref, ab_ref, choice_ref, value_ref):
        c0 = (zA_ref[:, 0:1] + p1a_ref[:, 0:1] + p1b_ref[:, 0:1]
              + 2.0 * (p2a_ref[:, 8:9] + p2b_ref[:, 8:9]) + b_ref[0, 0])
        valid = lax.broadcasted_iota(jnp.int32, (N2, 1), 0) < N
        c = jnp.where(valid, c0, -jnp.inf)
        m = jnp.max(c)
        ex = jnp.exp(c - m)
        choice_ref[...] = ex / jnp.sum(ex)
        v = jnp.sum(jnp.where(valid, h2_ref[...], 0.0), axis=0,
                    keepdims=True) * (1.0 / N)
        value_ref[...] = (
            jnp.sum(v * aw_ref[...], axis=1, keepdims=True) + ab_ref[...])

    return pl.pallas_call(
        body,
        out_shape=(jax.ShapeDtypeStruct((N2, 1), jnp.float32),
                   jax.ShapeDtypeStruct((1, 1), jnp.float32)),
    )(zA3, p1a, p1b, p2a, p2b, brow, h2, A2w, A2b)


def kernel(x, edge_index, weight, W1, b1, W2, b2, W3, b3, A2w, A2b):
    pad = EP - E
    src = jnp.pad(edge_index[0], (0, pad)).reshape(R, BLK)
    dst = jnp.pad(edge_index[1], (0, pad)).reshape(R, BLK)
    w2d = jnp.pad(weight, (0, pad)).reshape(R, BLK)

    W1cat = jnp.concatenate([W1[0] - W1[2], W1[1], W1[2]], axis=1)
    W2cat = jnp.concatenate([W2[0] - W2[2], W2[1], W2[2]], axis=1)
    W3p16 = jnp.pad(W3, ((0, 0), (0, 0), (0, 15)))
    W3p8 = jnp.pad(W3, ((0, 0), (0, 0), (0, 7)))
    W3cat = jnp.concatenate([W3p16[0] - W3p16[2], W3p8[1], W3p8[2]], axis=1)
    b1r = b1.reshape(1, 16)
    b2r = b2.reshape(1, 16)
    b3r = jnp.pad(b3, (0, 15)).reshape(1, 16)

    z1 = jnp.zeros((N2,), jnp.float32)
    z16 = jnp.zeros((N2, 16), jnp.float32)
    z32 = jnp.zeros((N2, 32), jnp.float32)

    deg_p = _deg_call(src, w2d, z1)
    xp = jnp.pad(x, ((0, N2 - N), (0, 0)))
    zA, zBC, dis1d = _tc_in(xp, W1cat, deg_p)

    p1, nn2d = _prop_call(zBC, src, dst, None, z32, 32, dis_w=(dis1d, w2d))
    p2 = _prop2_call(p1[0], p1[1], src, dst, nn2d, z16, 32)
    zA, zBC, _ = _tc_combine(zA, p1[0], p1[1], p2[0], p2[1], b1r, W2cat, 32)

    p1 = _prop_call(zBC, src, dst, nn2d, z32, 32)
    p2 = _prop2_call(p1[0], p1[1], src, dst, nn2d, z16, 32)
    zA, zBC, h2 = _tc_combine(zA, p1[0], p1[1], p2[0], p2[1], b2r, W3cat, 16)

    p1 = _prop_call(zBC, src, dst, nn2d, z16, 16)
    p2 = _prop2_call(p1[0], p1[1], src, dst, nn2d, z16, 16)
    choice, value = _tc_final(zA, p1[0], p1[1], p2[0], p2[1], b3r, h2,
                              A2w, A2b.reshape(1, 1))
    return choice[:N, 0], value.reshape(())
